# baseline (speedup 1.0000x reference)
import functools

import jax
import jax.numpy as jnp
from jax import lax
from jax.experimental import pallas as pl
from jax.experimental.pallas import tpu as pltpu
from jax.experimental.pallas import tpu_sc as plsc

_N_E = 8192
_E_DIM = 256
_BETA = 0.25
_B = 8
_L = 576
_N_TOK = _B * _L
_KB = 4096
_NB = _N_E // _KB

_NC = 2
_NS = 16
_NW = _NC * _NS
_TPW = _N_TOK // _NW
_TPW_A = 80
_TPW_B = 64


def _argmin_body(z_ref, emb_ref, ov_ref, oi_ref, ztm2_ref):
    kb = pl.program_id(0)

    @pl.when(kb == 0)
    def _():
        for b in range(_B):
            ztm2_ref[b] = -2.0 * jnp.transpose(z_ref[b])

    emb_blk = emb_ref[...]
    esq = jnp.transpose(
        jnp.sum(emb_blk * emb_blk, axis=1, keepdims=True))
    kidx = jax.lax.broadcasted_iota(jnp.int32, (_L, _KB), 1).astype(jnp.float32)
    for b in range(_B):
        ztb = ztm2_ref[b]
        a = 0.25 * jnp.sum(ztb * ztb, axis=1, keepdims=True)
        c2 = jax.lax.dot_general(
            ztb, emb_blk, (((1,), (1,)), ((), ())),
            preferred_element_type=jnp.float32)
        d = (a + esq) + c2
        mv = jnp.min(d, axis=1, keepdims=True)
        mi = jnp.min(jnp.where(d == mv, kidx, jnp.float32(1e9)),
                     axis=1, keepdims=True).astype(jnp.int32) + kb * _KB

        @pl.when(kb == 0)
        def _():
            ov_ref[:, b:b + 1] = mv
            oi_ref[:, b:b + 1] = mi

        @pl.when(kb != 0)
        def _():
            old_v = ov_ref[:, b:b + 1]
            upd = mv < old_v
            ov_ref[:, b:b + 1] = jnp.where(upd, mv, old_v)
            oi_ref[:, b:b + 1] = jnp.where(upd, mi, oi_ref[:, b:b + 1])


def _argmin_call(z, emb):
    return pl.pallas_call(
        _argmin_body,
        grid=(_NB,),
        in_specs=[
            pl.BlockSpec((_B, _E_DIM, _L), lambda kb: (0, 0, 0)),
            pl.BlockSpec((_KB, _E_DIM), lambda kb: (kb, 0)),
        ],
        out_specs=[
            pl.BlockSpec((_L, _B), lambda kb: (0, 0)),
            pl.BlockSpec((_L, _B), lambda kb: (0, 0)),
        ],
        out_shape=[
            jax.ShapeDtypeStruct((_L, _B), jnp.float32),
            jax.ShapeDtypeStruct((_L, _B), jnp.int32),
        ],
        scratch_shapes=[pltpu.VMEM((_B, _L, _E_DIM), jnp.float32)],
    )(z, emb)


_sc_mesh = plsc.VectorSubcoreMesh(core_axis_name="c", subcore_axis_name="s")


@functools.partial(
    pl.kernel,
    mesh=_sc_mesh,
    out_type=[
        jax.ShapeDtypeStruct((_N_TOK, _E_DIM), jnp.float32),
        jax.ShapeDtypeStruct((_NC, _N_E), jnp.float32),
    ],
    scratch_types=[
        pltpu.VMEM((_TPW_A,), jnp.int32),
        pltpu.VMEM((_TPW_B,), jnp.int32),
        pltpu.VMEM((_TPW_A, _E_DIM), jnp.float32),
        pltpu.VMEM((_TPW_B, _E_DIM), jnp.float32),
        pltpu.VMEM((_TPW_A,), jnp.float32),
        pltpu.VMEM((_TPW_B,), jnp.float32),
        pltpu.VMEM((_N_E // _NS,), jnp.float32),
        pltpu.VMEM_SHARED((_N_E,), jnp.float32),
        pltpu.SemaphoreType.DMA,
    ],
)
def _sc_gather_hist(emb_hbm, idx_hbm, q_hbm, parts_hbm,
                    idx_a, idx_b, rows_a, rows_b, ones_a, ones_b,
                    zeros_v, shared_counts, sem):
    cid = lax.axis_index("c")
    sid = lax.axis_index("s")
    wid = sid * _NC + cid
    base = wid * _TPW
    pltpu.sync_copy(idx_hbm.at[pl.ds(base, _TPW_A)], idx_a)
    pltpu.sync_copy(idx_hbm.at[pl.ds(base + _TPW_A, _TPW_B)], idx_b)
    cp_a = pltpu.async_copy(emb_hbm.at[idx_a], rows_a, sem)
    cp_b = pltpu.async_copy(emb_hbm.at[idx_b], rows_b, sem)
    one = jnp.ones((16,), jnp.float32)
    for j in range(_TPW_A // 16):
        ones_a[pl.ds(j * 16, 16)] = one
    for j in range(_TPW_B // 16):
        ones_b[pl.ds(j * 16, 16)] = one

    seg = _N_E // _NS
    for j in range(seg // 16):
        zeros_v[pl.ds(j * 16, 16)] = jnp.zeros((16,), jnp.float32)
    pltpu.sync_copy(zeros_v, shared_counts.at[pl.ds(sid * seg, seg)])

    plsc.subcore_barrier()
    pltpu.sync_copy(ones_a, shared_counts.at[idx_a], add=True)
    pltpu.sync_copy(ones_b, shared_counts.at[idx_b], add=True)
    plsc.subcore_barrier()

    @pl.when(sid == 0)
    def _():
        pltpu.sync_copy(shared_counts, parts_hbm.at[cid])

    cp_a.wait()
    cp_b.wait()
    pltpu.sync_copy(rows_a, q_hbm.at[pl.ds(base, _TPW_A)])
    pltpu.sync_copy(rows_b, q_hbm.at[pl.ds(base + _TPW_A, _TPW_B)])


def _epilogue_body(mv_ref, parts_ref, q_ref, loss_ref, ppl_ref, zq_ref):
    s = jnp.sum(mv_ref[...])
    loss = (1.0 + _BETA) * s / (_N_TOK * _E_DIM)
    loss_ref[...] = jnp.full((1, 1), loss, jnp.float32)
    counts = jnp.sum(parts_ref[...], axis=0, keepdims=True)
    e_mean = counts * (1.0 / _N_TOK)
    ent = jnp.sum(e_mean * jnp.log(e_mean + 1e-10))
    ppl_ref[...] = jnp.full((1, 1), jnp.exp(-ent), jnp.float32)
    for b in range(_B):
        zq_ref[b] = jnp.transpose(q_ref[b])


def _epilogue_call(minval, parts, quantized):
    return pl.pallas_call(
        _epilogue_body,
        out_shape=[
            jax.ShapeDtypeStruct((1, 1), jnp.float32),
            jax.ShapeDtypeStruct((1, 1), jnp.float32),
            jax.ShapeDtypeStruct((_B, _E_DIM, _L), jnp.float32),
        ],
    )(minval, parts, quantized)


def kernel(z, emb):
    minval, minidx = _argmin_call(z, emb)
    idx = minidx.T.reshape(-1)
    quantized, parts = _sc_gather_hist(emb, idx)
    loss2d, ppl2d, z_q = _epilogue_call(
        minval, parts, quantized.reshape(_B, _L, _E_DIM))
    return (loss2d[0, 0], z_q, ppl2d[0, 0], idx[:, None])
